# baseline (device time: 28223 ns/iter reference)
import jax
import jax.numpy as jnp
from jax import lax
from jax.experimental import pallas as pl
from jax.experimental.pallas import tpu as pltpu

N_DEV = 4
N_SC = 4

S_DIRECT, S_FAR, S_COMB, S_AGOWN = 0, 1, 2, 3
R_DIRECT, R_FAR, R_COMB, R_AG_L, R_AG_R, R_AGFAR = 4, 5, 6, 7, 8, 9
N_SLOT = 10

T_DIRECT, T_FAR, T_COMB, T_AG_R, T_AG_L, T_AGFAR = 0, 1, 2, 3, 4, 5
N_TYPE = 6


def kernel(A, B):
    m, k = A.shape
    _, n = B.shape
    ch = m // N_DEV
    sub = ch // N_SC
    half = n // 2

    f32 = jnp.float32
    bf16 = jnp.bfloat16

    def body(a_ref, b_ref, out_ref, p_ref, comm, send_sems, recv_sems):
        my = lax.axis_index("i")
        left = (my - 1) % N_DEV
        right = (my + 1) % N_DEV

        barrier_sem = pltpu.get_barrier_semaphore()
        for nbr in [left, right]:
            pl.semaphore_signal(
                barrier_sem, inc=1,
                device_id=(nbr,), device_id_type=pl.DeviceIdType.MESH,
            )
        pl.semaphore_wait(barrier_sem, 2)

        def rows(c):
            return pl.ds((c % N_DEV) * ch, ch)

        def rows_sc(c, h):
            return pl.ds((c % N_DEV) * ch + h * sub, sub)

        def cols(d):
            return pl.ds(d * half, half)

        def rdma(d, t, h, src_slot, dst_slot, to_right):
            return pltpu.make_async_remote_copy(
                src_ref=comm.at[d, src_slot, h],
                dst_ref=comm.at[d, dst_slot, h],
                send_sem=send_sems.at[d, t, h],
                recv_sem=recv_sems.at[d, t, h],
                device_id=(right if to_right else left,),
                device_id_type=pl.DeviceIdType.MESH,
            )

        dirs = {}
        for d in (0, 1):
            r = d == 0
            for h in range(N_SC):
                dirs[(d, T_DIRECT, h)] = rdma(d, T_DIRECT, h, S_DIRECT,
                                              R_DIRECT, r)
                dirs[(d, T_FAR, h)] = rdma(d, T_FAR, h, S_FAR, R_FAR, not r)
                dirs[(d, T_COMB, h)] = rdma(d, T_COMB, h, S_COMB, R_COMB,
                                            not r)
                dirs[(d, T_AG_R, h)] = rdma(d, T_AG_R, h, S_AGOWN, R_AG_L,
                                            True)
                dirs[(d, T_AG_L, h)] = rdma(d, T_AG_L, h, S_AGOWN, R_AG_R,
                                            False)
                dirs[(d, T_AGFAR, h)] = rdma(
                    d, T_AGFAR, h, R_AG_L if d == 0 else R_AG_R, R_AGFAR, r)

        def dot_block(c):
            p_ref[rows(c), :] = jnp.dot(
                a_ref[rows(c), :], b_ref[...], preferred_element_type=f32)

        def stage_and_send(d, t, slot, c, h):
            comm[d, slot, h] = p_ref[rows_sc(c, h), cols(d)].astype(bf16)
            dirs[(d, t, h)].start()

        dot_block(my + 2)
        for h in range(N_SC):
            stage_and_send(0, T_FAR, S_FAR, my + 2, h)
            stage_and_send(1, T_FAR, S_FAR, my + 2, h)

        dot_block(my + 1)
        stage_and_send(0, T_DIRECT, S_DIRECT, my + 1, 0)
        dot_block(my - 1)
        stage_and_send(1, T_DIRECT, S_DIRECT, my - 1, 0)

        relay_chunk = {0: my - 1, 1: my + 1}
        for h in range(N_SC):
            for d in (0, 1):
                dirs[(d, T_FAR, h)].wait_recv()
                comm[d, S_COMB, h] = (
                    comm[d, R_FAR, h].astype(f32)
                    + p_ref[rows_sc(relay_chunk[d], h), cols(d)]
                ).astype(bf16)
                dirs[(d, T_COMB, h)].start()
            if h + 1 < N_SC:
                stage_and_send(0, T_DIRECT, S_DIRECT, my + 1, h + 1)
                stage_and_send(1, T_DIRECT, S_DIRECT, my - 1, h + 1)
            if h == 0:
                dot_block(my)

        for h in range(N_SC):
            for d in (0, 1):
                dirs[(d, T_DIRECT, h)].wait_recv()
                dirs[(d, T_COMB, h)].wait_recv()
                full = jnp.maximum(
                    p_ref[rows_sc(my, h), cols(d)]
                    + comm[d, R_DIRECT, h].astype(f32)
                    + comm[d, R_COMB, h].astype(f32), 0.0)
                comm[d, S_AGOWN, h] = full.astype(bf16)
                dirs[(d, T_AG_R, h)].start()
                dirs[(d, T_AG_L, h)].start()
                out_ref[rows_sc(my, h), cols(d)] = full

        for h in range(N_SC):
            dirs[(0, T_AG_R, h)].wait_recv()
            dirs[(0, T_AGFAR, h)].start()
            out_ref[rows_sc(my - 1, h), cols(0)] = (
                comm[0, R_AG_L, h].astype(f32))

            dirs[(1, T_AG_L, h)].wait_recv()
            dirs[(1, T_AGFAR, h)].start()
            out_ref[rows_sc(my + 1, h), cols(1)] = (
                comm[1, R_AG_R, h].astype(f32))

        for h in range(N_SC):
            dirs[(0, T_AG_L, h)].wait_recv()
            out_ref[rows_sc(my + 1, h), cols(0)] = (
                comm[0, R_AG_R, h].astype(f32))
            dirs[(1, T_AG_R, h)].wait_recv()
            out_ref[rows_sc(my - 1, h), cols(1)] = (
                comm[1, R_AG_L, h].astype(f32))

        for h in range(N_SC):
            dirs[(0, T_AGFAR, h)].wait_recv()
            out_ref[rows_sc(my + 2, h), cols(0)] = (
                comm[0, R_AGFAR, h].astype(f32))
            dirs[(1, T_AGFAR, h)].wait_recv()
            out_ref[rows_sc(my + 2, h), cols(1)] = (
                comm[1, R_AGFAR, h].astype(f32))

        for r in dirs.values():
            r.wait_send()

    return pl.pallas_call(
        body,
        out_shape=jax.ShapeDtypeStruct((m, n), f32),
        in_specs=[
            pl.BlockSpec(memory_space=pltpu.VMEM),
            pl.BlockSpec(memory_space=pltpu.VMEM),
        ],
        out_specs=pl.BlockSpec(memory_space=pltpu.VMEM),
        scratch_shapes=[
            pltpu.VMEM((m, n), f32),
            pltpu.VMEM((2, N_SLOT, N_SC, sub, half), bf16),
            pltpu.SemaphoreType.DMA((2, N_TYPE, N_SC)),
            pltpu.SemaphoreType.DMA((2, N_TYPE, N_SC)),
        ],
        compiler_params=pltpu.CompilerParams(collective_id=0),
    )(A, B)


# device time: 27971 ns/iter; 1.0090x vs baseline; 1.0090x over previous
import jax
import jax.numpy as jnp
from jax import lax
from jax.experimental import pallas as pl
from jax.experimental.pallas import tpu as pltpu

N_DEV = 4
N_SC = 2

S_DIRECT, S_FAR, S_COMB, S_AGOWN = 0, 1, 2, 3
R_DIRECT, R_FAR, R_COMB, R_AG_L, R_AG_R, R_AGFAR = 4, 5, 6, 7, 8, 9
N_SLOT = 10

T_DIRECT, T_FAR, T_COMB, T_AG_R, T_AG_L, T_AGFAR = 0, 1, 2, 3, 4, 5
N_TYPE = 6


def kernel(A, B):
    m, k = A.shape
    _, n = B.shape
    ch = m // N_DEV
    sub = ch // N_SC
    half = n // 2

    f32 = jnp.float32
    bf16 = jnp.bfloat16

    def body(a_ref, b_ref, out_ref, p_ref, comm, send_sems, recv_sems):
        my = lax.axis_index("i")
        left = (my - 1) % N_DEV
        right = (my + 1) % N_DEV

        barrier_sem = pltpu.get_barrier_semaphore()
        for nbr in [left, right]:
            pl.semaphore_signal(
                barrier_sem, inc=1,
                device_id=(nbr,), device_id_type=pl.DeviceIdType.MESH,
            )
        pl.semaphore_wait(barrier_sem, 2)

        def rows(c):
            return pl.ds((c % N_DEV) * ch, ch)

        def rows_sc(c, h):
            return pl.ds((c % N_DEV) * ch + h * sub, sub)

        def cols(d):
            return pl.ds(d * half, half)

        def rdma(d, t, h, src_slot, dst_slot, to_right):
            return pltpu.make_async_remote_copy(
                src_ref=comm.at[d, src_slot, h],
                dst_ref=comm.at[d, dst_slot, h],
                send_sem=send_sems.at[d, t, h],
                recv_sem=recv_sems.at[d, t, h],
                device_id=(right if to_right else left,),
                device_id_type=pl.DeviceIdType.MESH,
            )

        dirs = {}
        for d in (0, 1):
            r = d == 0
            for h in range(N_SC):
                dirs[(d, T_DIRECT, h)] = rdma(d, T_DIRECT, h, S_DIRECT,
                                              R_DIRECT, r)
                dirs[(d, T_FAR, h)] = rdma(d, T_FAR, h, S_FAR, R_FAR, not r)
                dirs[(d, T_COMB, h)] = rdma(d, T_COMB, h, S_COMB, R_COMB,
                                            not r)
                dirs[(d, T_AG_R, h)] = rdma(d, T_AG_R, h, S_AGOWN, R_AG_L,
                                            True)
                dirs[(d, T_AG_L, h)] = rdma(d, T_AG_L, h, S_AGOWN, R_AG_R,
                                            False)
                dirs[(d, T_AGFAR, h)] = rdma(
                    d, T_AGFAR, h, R_AG_L if d == 0 else R_AG_R, R_AGFAR, r)

        def dot_block(c):
            p_ref[rows(c), :] = jnp.dot(
                a_ref[rows(c), :], b_ref[...], preferred_element_type=f32)

        def dot_block_half(c, d):
            p_ref[rows(c), cols(d)] = jnp.dot(
                a_ref[rows(c), :], b_ref[:, cols(d)],
                preferred_element_type=f32)

        def stage_and_send(d, t, slot, c, h):
            comm[d, slot, h] = p_ref[rows_sc(c, h), cols(d)].astype(bf16)
            dirs[(d, t, h)].start()

        dot_block_half(my + 2, 0)
        for h in range(N_SC):
            stage_and_send(0, T_FAR, S_FAR, my + 2, h)
        dot_block_half(my + 2, 1)
        for h in range(N_SC):
            stage_and_send(1, T_FAR, S_FAR, my + 2, h)

        dot_block_half(my + 1, 0)
        stage_and_send(0, T_DIRECT, S_DIRECT, my + 1, 0)
        dot_block_half(my - 1, 1)
        stage_and_send(1, T_DIRECT, S_DIRECT, my - 1, 0)
        dot_block_half(my - 1, 0)
        dot_block_half(my + 1, 1)

        relay_chunk = {0: my - 1, 1: my + 1}
        for h in range(N_SC):
            for d in (0, 1):
                dirs[(d, T_FAR, h)].wait_recv()
                comm[d, S_COMB, h] = (
                    comm[d, R_FAR, h].astype(f32)
                    + p_ref[rows_sc(relay_chunk[d], h), cols(d)]
                ).astype(bf16)
                dirs[(d, T_COMB, h)].start()
            if h + 1 < N_SC:
                stage_and_send(0, T_DIRECT, S_DIRECT, my + 1, h + 1)
                stage_and_send(1, T_DIRECT, S_DIRECT, my - 1, h + 1)
            if h == 0:
                dot_block(my)

        for h in range(N_SC):
            for d in (0, 1):
                dirs[(d, T_DIRECT, h)].wait_recv()
                dirs[(d, T_COMB, h)].wait_recv()
                full = jnp.maximum(
                    p_ref[rows_sc(my, h), cols(d)]
                    + comm[d, R_DIRECT, h].astype(f32)
                    + comm[d, R_COMB, h].astype(f32), 0.0)
                comm[d, S_AGOWN, h] = full.astype(bf16)
                dirs[(d, T_AG_R, h)].start()
                dirs[(d, T_AG_L, h)].start()
                out_ref[rows_sc(my, h), cols(d)] = full

        for h in range(N_SC):
            dirs[(0, T_AG_R, h)].wait_recv()
            dirs[(0, T_AGFAR, h)].start()
            out_ref[rows_sc(my - 1, h), cols(0)] = (
                comm[0, R_AG_L, h].astype(f32))

            dirs[(1, T_AG_L, h)].wait_recv()
            dirs[(1, T_AGFAR, h)].start()
            out_ref[rows_sc(my + 1, h), cols(1)] = (
                comm[1, R_AG_R, h].astype(f32))

        for h in range(N_SC):
            dirs[(0, T_AG_L, h)].wait_recv()
            out_ref[rows_sc(my + 1, h), cols(0)] = (
                comm[0, R_AG_R, h].astype(f32))
            dirs[(1, T_AG_R, h)].wait_recv()
            out_ref[rows_sc(my - 1, h), cols(1)] = (
                comm[1, R_AG_L, h].astype(f32))

        for h in range(N_SC):
            dirs[(0, T_AGFAR, h)].wait_recv()
            out_ref[rows_sc(my + 2, h), cols(0)] = (
                comm[0, R_AGFAR, h].astype(f32))
            dirs[(1, T_AGFAR, h)].wait_recv()
            out_ref[rows_sc(my + 2, h), cols(1)] = (
                comm[1, R_AGFAR, h].astype(f32))

        for r in dirs.values():
            r.wait_send()

    return pl.pallas_call(
        body,
        out_shape=jax.ShapeDtypeStruct((m, n), f32),
        in_specs=[
            pl.BlockSpec(memory_space=pltpu.VMEM),
            pl.BlockSpec(memory_space=pltpu.VMEM),
        ],
        out_specs=pl.BlockSpec(memory_space=pltpu.VMEM),
        scratch_shapes=[
            pltpu.VMEM((m, n), f32),
            pltpu.VMEM((2, N_SLOT, N_SC, sub, half), bf16),
            pltpu.SemaphoreType.DMA((2, N_TYPE, N_SC)),
            pltpu.SemaphoreType.DMA((2, N_TYPE, N_SC)),
        ],
        compiler_params=pltpu.CompilerParams(collective_id=0),
    )(A, B)


# device time: 27738 ns/iter; 1.0175x vs baseline; 1.0084x over previous
import jax
import jax.numpy as jnp
from jax import lax
from jax.experimental import pallas as pl
from jax.experimental.pallas import tpu as pltpu

N_DEV = 4
N_Q = 4

S_DIRECT, S_FAR, S_COMB, S_AGOWN = 0, 1, 2, 3
R_DIRECT, R_FAR, R_COMB, R_AG_L, R_AG_R, R_AGFAR = 4, 5, 6, 7, 8, 9
N_SLOT = 10

T_DIRECT, T_FAR, T_COMB, T_AG_R, T_AG_L, T_AGFAR = 0, 1, 2, 3, 4, 5
N_TYPE = 6


def kernel(A, B):
    m, k = A.shape
    _, n = B.shape
    ch = m // N_DEV
    qw = n // N_Q

    f32 = jnp.float32
    bf16 = jnp.bfloat16

    QDIR = (0, 0, 1, 1)

    def body(a_ref, b_ref, out_ref, p_ref, comm, send_sems, recv_sems):
        my = lax.axis_index("i")
        left = (my - 1) % N_DEV
        right = (my + 1) % N_DEV

        barrier_sem = pltpu.get_barrier_semaphore()
        for nbr in [left, right]:
            pl.semaphore_signal(
                barrier_sem, inc=1,
                device_id=(nbr,), device_id_type=pl.DeviceIdType.MESH,
            )
        pl.semaphore_wait(barrier_sem, 2)

        def rows(c):
            return pl.ds((c % N_DEV) * ch, ch)

        def cols(q):
            return pl.ds(q * qw, qw)

        def rdma(q, t, src_slot, dst_slot, to_right):
            return pltpu.make_async_remote_copy(
                src_ref=comm.at[q, src_slot],
                dst_ref=comm.at[q, dst_slot],
                send_sem=send_sems.at[q, t],
                recv_sem=recv_sems.at[q, t],
                device_id=(right if to_right else left,),
                device_id_type=pl.DeviceIdType.MESH,
            )

        dirs = {}
        for q in range(N_Q):
            r = QDIR[q] == 0
            dirs[(q, T_DIRECT)] = rdma(q, T_DIRECT, S_DIRECT, R_DIRECT, r)
            dirs[(q, T_FAR)] = rdma(q, T_FAR, S_FAR, R_FAR, not r)
            dirs[(q, T_COMB)] = rdma(q, T_COMB, S_COMB, R_COMB, not r)
            dirs[(q, T_AG_R)] = rdma(q, T_AG_R, S_AGOWN, R_AG_L, True)
            dirs[(q, T_AG_L)] = rdma(q, T_AG_L, S_AGOWN, R_AG_R, False)
            dirs[(q, T_AGFAR)] = rdma(
                q, T_AGFAR, R_AG_L if r else R_AG_R, R_AGFAR, r)

        def dot_block(c):
            p_ref[rows(c), :] = jnp.dot(
                a_ref[rows(c), :], b_ref[...], preferred_element_type=f32)

        def stage_and_send(q, t, slot, c):
            comm[q, slot] = p_ref[rows(c), cols(q)].astype(bf16)
            dirs[(q, t)].start()

        def direct_chunk(q):
            return my + 1 if QDIR[q] == 0 else my - 1

        def relay_chunk(q):
            return my - 1 if QDIR[q] == 0 else my + 1

        dot_block(my + 2)
        for q in (0, 2, 1, 3):
            stage_and_send(q, T_FAR, S_FAR, my + 2)

        dot_block(my + 1)
        stage_and_send(0, T_DIRECT, S_DIRECT, my + 1)
        dot_block(my - 1)
        stage_and_send(2, T_DIRECT, S_DIRECT, my - 1)
        dot_block(my)

        for qs in ((0, 2), (1, 3)):
            for q in qs:
                dirs[(q, T_FAR)].wait_recv()
                comm[q, S_COMB] = (
                    comm[q, R_FAR].astype(f32)
                    + p_ref[rows(relay_chunk(q)), cols(q)]
                ).astype(bf16)
                dirs[(q, T_COMB)].start()
            if qs == (0, 2):
                stage_and_send(1, T_DIRECT, S_DIRECT, my + 1)
                stage_and_send(3, T_DIRECT, S_DIRECT, my - 1)

        ag_full = {}
        for qs, feeds_only in (((0, 2), False), ((1, 3), True)):
            for q in qs:
                dirs[(q, T_DIRECT)].wait_recv()
                dirs[(q, T_COMB)].wait_recv()
                full = jnp.maximum(
                    p_ref[rows(my), cols(q)]
                    + comm[q, R_DIRECT].astype(f32)
                    + comm[q, R_COMB].astype(f32), 0.0)
                comm[q, S_AGOWN] = full.astype(bf16)
                ag_full[q] = full
            for q in qs:
                dirs[(q, T_AG_R if QDIR[q] == 0 else T_AG_L)].start()
            if not feeds_only:
                for q in qs:
                    dirs[(q, T_AG_L if QDIR[q] == 0 else T_AG_R)].start()
            for q in qs:
                out_ref[rows(my), cols(q)] = ag_full[q]

        for q in (0, 2):
            src = R_AG_L if QDIR[q] == 0 else R_AG_R
            dirs[(q, T_AG_R if QDIR[q] == 0 else T_AG_L)].wait_recv()
            dirs[(q, T_AGFAR)].start()
            out_ref[rows(relay_chunk(q)), cols(q)] = comm[q, src].astype(f32)

        for q in (1, 3):
            dirs[(q, T_AG_L if QDIR[q] == 0 else T_AG_R)].start()

        for q in (1, 3):
            src = R_AG_L if QDIR[q] == 0 else R_AG_R
            dirs[(q, T_AG_R if QDIR[q] == 0 else T_AG_L)].wait_recv()
            dirs[(q, T_AGFAR)].start()
            out_ref[rows(relay_chunk(q)), cols(q)] = comm[q, src].astype(f32)

        for q in (0, 2, 1, 3):
            src = R_AG_R if QDIR[q] == 0 else R_AG_L
            dirs[(q, T_AG_L if QDIR[q] == 0 else T_AG_R)].wait_recv()
            out_ref[rows(direct_chunk(q)), cols(q)] = comm[q, src].astype(f32)
        for q in (0, 2, 1, 3):
            dirs[(q, T_AGFAR)].wait_recv()
            out_ref[rows(my + 2), cols(q)] = comm[q, R_AGFAR].astype(f32)

        for r in dirs.values():
            r.wait_send()

    return pl.pallas_call(
        body,
        out_shape=jax.ShapeDtypeStruct((m, n), f32),
        in_specs=[
            pl.BlockSpec(memory_space=pltpu.VMEM),
            pl.BlockSpec(memory_space=pltpu.VMEM),
        ],
        out_specs=pl.BlockSpec(memory_space=pltpu.VMEM),
        scratch_shapes=[
            pltpu.VMEM((m, n), f32),
            pltpu.VMEM((N_Q, N_SLOT, ch, qw), bf16),
            pltpu.SemaphoreType.DMA((N_Q, N_TYPE)),
            pltpu.SemaphoreType.DMA((N_Q, N_TYPE)),
        ],
        compiler_params=pltpu.CompilerParams(collective_id=0),
    )(A, B)


# device time: 27153 ns/iter; 1.0394x vs baseline; 1.0215x over previous
import jax
import jax.numpy as jnp
from jax import lax
from jax.experimental import pallas as pl
from jax.experimental.pallas import tpu as pltpu

N_DEV = 4
N_Q = 4

S_DIRECT, S_FAR, S_COMB, S_AGOWN = 0, 1, 2, 3
R_DIRECT, R_FAR, R_COMB, R_AG_L, R_AG_R, R_AGFAR = 4, 5, 6, 7, 8, 9
N_SLOT = 10

T_DIRECT, T_FAR, T_COMB, T_AG_R, T_AG_L, T_AGFAR = 0, 1, 2, 3, 4, 5
N_TYPE = 6


def kernel(A, B):
    m, k = A.shape
    _, n = B.shape
    ch = m // N_DEV
    qw = n // N_Q

    f32 = jnp.float32
    bf16 = jnp.bfloat16

    QDIR = (0, 0, 1, 1)

    def body(a_ref, b_ref, out_ref, a_vmem, b_vmem, out_stage, p_ref, comm,
             send_sems, recv_sems, in_sems, out_dma_sems):
        my = lax.axis_index("i")
        left = (my - 1) % N_DEV
        right = (my + 1) % N_DEV

        copy_b = pltpu.make_async_copy(b_ref, b_vmem, in_sems.at[1])
        copy_b.start()
        copy_a = pltpu.make_async_copy(a_ref, a_vmem, in_sems.at[0])
        copy_a.start()

        barrier_sem = pltpu.get_barrier_semaphore()
        for nbr in [left, right]:
            pl.semaphore_signal(
                barrier_sem, inc=1,
                device_id=(nbr,), device_id_type=pl.DeviceIdType.MESH,
            )
        pl.semaphore_wait(barrier_sem, 2)
        copy_b.wait()
        copy_a.wait()

        def rows(c):
            return pl.ds((c % N_DEV) * ch, ch)

        def cols(q):
            return pl.ds(q * qw, qw)

        def rdma(q, t, src_slot, dst_slot, to_right):
            return pltpu.make_async_remote_copy(
                src_ref=comm.at[q, src_slot],
                dst_ref=comm.at[q, dst_slot],
                send_sem=send_sems.at[q, t],
                recv_sem=recv_sems.at[q, t],
                device_id=(right if to_right else left,),
                device_id_type=pl.DeviceIdType.MESH,
            )

        dirs = {}
        for q in range(N_Q):
            r = QDIR[q] == 0
            dirs[(q, T_DIRECT)] = rdma(q, T_DIRECT, S_DIRECT, R_DIRECT, r)
            dirs[(q, T_FAR)] = rdma(q, T_FAR, S_FAR, R_FAR, not r)
            dirs[(q, T_COMB)] = rdma(q, T_COMB, S_COMB, R_COMB, not r)
            dirs[(q, T_AG_R)] = rdma(q, T_AG_R, S_AGOWN, R_AG_L, True)
            dirs[(q, T_AG_L)] = rdma(q, T_AG_L, S_AGOWN, R_AG_R, False)
            dirs[(q, T_AGFAR)] = rdma(
                q, T_AGFAR, R_AG_L if r else R_AG_R, R_AGFAR, r)

        def dot_block(c):
            p_ref[rows(c), :] = jnp.dot(
                a_vmem[rows(c), :], b_vmem[...], preferred_element_type=f32)

        def stage_and_send(q, t, slot, c):
            comm[q, slot] = p_ref[rows(c), cols(q)].astype(bf16)
            dirs[(q, t)].start()

        out_copies = []

        def put_out(c, q, kind, values):
            out_stage[rows(c), cols(q)] = values
            cp = pltpu.make_async_copy(
                out_stage.at[rows(c), cols(q)],
                out_ref.at[rows(c), cols(q)],
                out_dma_sems.at[q, kind],
            )
            cp.start()
            out_copies.append(cp)

        def direct_chunk(q):
            return my + 1 if QDIR[q] == 0 else my - 1

        def relay_chunk(q):
            return my - 1 if QDIR[q] == 0 else my + 1

        dot_block(my + 2)
        for q in (0, 2, 1, 3):
            stage_and_send(q, T_FAR, S_FAR, my + 2)

        dot_block(my + 1)
        stage_and_send(0, T_DIRECT, S_DIRECT, my + 1)
        dot_block(my - 1)
        stage_and_send(2, T_DIRECT, S_DIRECT, my - 1)
        dot_block(my)

        for qs in ((0, 2), (1, 3)):
            for q in qs:
                dirs[(q, T_FAR)].wait_recv()
                comm[q, S_COMB] = (
                    comm[q, R_FAR].astype(f32)
                    + p_ref[rows(relay_chunk(q)), cols(q)]
                ).astype(bf16)
                dirs[(q, T_COMB)].start()
            if qs == (0, 2):
                stage_and_send(1, T_DIRECT, S_DIRECT, my + 1)
                stage_and_send(3, T_DIRECT, S_DIRECT, my - 1)

        ag_full = {}
        for qs, feeds_only in (((0, 2), False), ((1, 3), True)):
            for q in qs:
                dirs[(q, T_DIRECT)].wait_recv()
                dirs[(q, T_COMB)].wait_recv()
                full = jnp.maximum(
                    p_ref[rows(my), cols(q)]
                    + comm[q, R_DIRECT].astype(f32)
                    + comm[q, R_COMB].astype(f32), 0.0)
                comm[q, S_AGOWN] = full.astype(bf16)
                ag_full[q] = full
            for q in qs:
                dirs[(q, T_AG_R if QDIR[q] == 0 else T_AG_L)].start()
            if not feeds_only:
                for q in qs:
                    dirs[(q, T_AG_L if QDIR[q] == 0 else T_AG_R)].start()
            for q in qs:
                put_out(my, q, 0, ag_full[q])

        for q in (0, 2):
            src = R_AG_L if QDIR[q] == 0 else R_AG_R
            dirs[(q, T_AG_R if QDIR[q] == 0 else T_AG_L)].wait_recv()
            dirs[(q, T_AGFAR)].start()
            put_out(relay_chunk(q), q, 1, comm[q, src].astype(f32))

        for q in (1, 3):
            dirs[(q, T_AG_L if QDIR[q] == 0 else T_AG_R)].start()

        for q in (1, 3):
            src = R_AG_L if QDIR[q] == 0 else R_AG_R
            dirs[(q, T_AG_R if QDIR[q] == 0 else T_AG_L)].wait_recv()
            dirs[(q, T_AGFAR)].start()
            put_out(relay_chunk(q), q, 1, comm[q, src].astype(f32))

        for q in (0, 2, 1, 3):
            src = R_AG_R if QDIR[q] == 0 else R_AG_L
            dirs[(q, T_AG_L if QDIR[q] == 0 else T_AG_R)].wait_recv()
            put_out(direct_chunk(q), q, 2, comm[q, src].astype(f32))
        for q in (0, 2, 1, 3):
            dirs[(q, T_AGFAR)].wait_recv()
            put_out(my + 2, q, 3, comm[q, R_AGFAR].astype(f32))

        for r in dirs.values():
            r.wait_send()
        for cp in out_copies:
            cp.wait()

    return pl.pallas_call(
        body,
        out_shape=jax.ShapeDtypeStruct((m, n), f32),
        in_specs=[
            pl.BlockSpec(memory_space=pl.ANY),
            pl.BlockSpec(memory_space=pl.ANY),
        ],
        out_specs=pl.BlockSpec(memory_space=pl.ANY),
        scratch_shapes=[
            pltpu.VMEM((m, k), f32),
            pltpu.VMEM((k, n), f32),
            pltpu.VMEM((m, n), f32),
            pltpu.VMEM((m, n), f32),
            pltpu.VMEM((N_Q, N_SLOT, ch, qw), bf16),
            pltpu.SemaphoreType.DMA((N_Q, N_TYPE)),
            pltpu.SemaphoreType.DMA((N_Q, N_TYPE)),
            pltpu.SemaphoreType.DMA((2,)),
            pltpu.SemaphoreType.DMA((N_Q, 4)),
        ],
        compiler_params=pltpu.CompilerParams(collective_id=0),
    )(A, B)
